# baseline (speedup 1.0000x reference)
import functools

import jax
import jax.numpy as jnp
import numpy as np
from jax.experimental import pallas as pl
from jax.experimental.pallas import tpu as pltpu

_TINY = 1.1754943508222875e-38


def _softplus(x):
    return jnp.maximum(x, 0.0) + jnp.log1p(jnp.exp(-jnp.abs(x)))


def _unary_body(x_ref, w_ref, s_ref, o_ref):
    x = x_ref[...]
    z = jnp.clip(x[:, :, :16], 0.0, 1.0)
    Z = jnp.clip(x[:, :, 16:], 0.0, 1.0)
    lv = jnp.sum(jnp.log(_softplus(Z - z) + _TINY), axis=2)
    t = lv + w_ref[...][:, :1]
    m0 = jnp.max(t, axis=0, keepdims=True)
    res = jnp.log(jnp.sum(jnp.exp(t - m0), axis=0)) + m0[0] - s_ref[0]
    o_ref[...] = res[None, None]


def _pair_body(idx_ref, xa_ref, xb_ref, w_ref, s_ref,
               a_ref, b_ref, p_ref, e_ref):
    xa = xa_ref[...][:, 0]
    xb = xb_ref[...][:, 0]
    ca = jnp.clip(xa, 0.0, 1.0)
    cb = jnp.clip(xb, 0.0, 1.0)
    a_ref[...] = ca[:, None]
    b_ref[...] = cb[:, None]
    wadj = w_ref[...][:, 0]
    di = jnp.minimum(ca[:, 1, :], cb[:, 1, :]) - jnp.maximum(ca[:, 0, :], cb[:, 0, :])
    db = cb[:, 1, :] - cb[:, 0, :]
    lvi = jnp.sum(jnp.log(_softplus(di) + _TINY), axis=1) + wadj
    lvb = jnp.sum(jnp.log(_softplus(db) + _TINY), axis=1) + wadj
    mi = jnp.max(lvi)
    mb = jnp.max(lvb)
    log_p = (jnp.log(jnp.sum(jnp.exp(lvi - mi))) + mi) - (
        jnp.log(jnp.sum(jnp.exp(lvb - mb))) + mb)
    p_ref[...] = jnp.broadcast_to(log_p, (1, 1, 8))
    e_ref[...] = jnp.broadcast_to(jnp.exp(log_p), (1, 1, 8))


def kernel(box_indices, box_param, weights):
    M, N, _, D = box_param.shape
    Bsz = box_indices.shape[0]
    NB = 2000
    grid1 = N // NB

    log_universe_vol = float(D) * float(np.log(np.log1p(np.e) + _TINY))
    wadj = (weights - log_universe_vol).astype(jnp.float32)
    wcol = jnp.broadcast_to(wadj[:, None], (M, 128))
    wmax = jnp.max(weights)
    lsew = (jnp.log(jnp.sum(jnp.exp(weights - wmax))) + wmax).reshape(1)

    table32 = box_param.reshape(M, N, 2 * D)

    unary2d = pl.pallas_call(
        _unary_body,
        grid=(grid1,),
        in_specs=[
            pl.BlockSpec((M, NB, 2 * D), lambda i: (0, i, 0)),
            pl.BlockSpec((M, 128), lambda i: (0, 0)),
            pl.BlockSpec(memory_space=pltpu.SMEM),
        ],
        out_specs=pl.BlockSpec((1, 1, NB), lambda i: (i, 0, 0)),
        out_shape=jax.ShapeDtypeStruct((grid1, 1, NB), jnp.float32),
    )(table32, wcol, lsew)
    log_unary_probs = unary2d.reshape(N)

    pair_out = pl.pallas_call(
        _pair_body,
        grid_spec=pltpu.PrefetchScalarGridSpec(
            num_scalar_prefetch=1,
            grid=(Bsz,),
            in_specs=[
                pl.BlockSpec((M, 1, 2, D), lambda b, idx: (0, idx[2 * b], 0, 0)),
                pl.BlockSpec((M, 1, 2, D), lambda b, idx: (0, idx[2 * b + 1], 0, 0)),
                pl.BlockSpec((M, 128), lambda b, idx: (0, 0)),
                pl.BlockSpec(memory_space=pltpu.SMEM),
            ],
            out_specs=[
                pl.BlockSpec((M, 1, 2, D), lambda b, idx: (0, b, 0, 0)),
                pl.BlockSpec((M, 1, 2, D), lambda b, idx: (0, b, 0, 0)),
                pl.BlockSpec((1, 1, 8), lambda b, idx: (b, 0, 0)),
                pl.BlockSpec((1, 1, 8), lambda b, idx: (b, 0, 0)),
            ],
        ),
        out_shape=[
            jax.ShapeDtypeStruct((M, Bsz, 2, D), jnp.float32),
            jax.ShapeDtypeStruct((M, Bsz, 2, D), jnp.float32),
            jax.ShapeDtypeStruct((Bsz, 1, 8), jnp.float32),
            jax.ShapeDtypeStruct((Bsz, 1, 8), jnp.float32),
        ],
    )(box_indices.reshape(-1), box_param, box_param, wcol, lsew)
    A, B, logp3, expp3 = pair_out
    log_p = logp3[:, 0, 0]
    exp_p = expp3[:, 0, 0]

    return (log_unary_probs, box_param, A, B, log_p, exp_p)


# speedup vs baseline: 7.3445x; 7.3445x over previous
import functools

import jax
import jax.numpy as jnp
import numpy as np
from jax import lax
from jax.experimental import pallas as pl
from jax.experimental.pallas import tpu as pltpu
from jax.experimental.pallas import tpu_sc as plsc

_TINY = 1.1754943508222875e-38


def _softplus(x):
    return jnp.maximum(x, 0.0) + jnp.log1p(jnp.exp(-jnp.abs(x)))


def _unary_body(x_ref, w_ref, s_ref, o_ref):
    x = x_ref[...]
    z = jnp.clip(x[:, :, :16], 0.0, 1.0)
    Z = jnp.clip(x[:, :, 16:], 0.0, 1.0)
    lv = jnp.sum(jnp.log(_softplus(Z - z) + _TINY), axis=2)
    t = lv + w_ref[...][:, :1]
    m0 = jnp.max(t, axis=0, keepdims=True)
    res = jnp.log(jnp.sum(jnp.exp(t - m0), axis=0)) + m0[0] - s_ref[0]
    o_ref[...] = res[None, None]


def _pair_body(dp_ref, w_ref, p_ref, e_ref):
    dp = dp_ref[...]
    lsp = jnp.log(_softplus(dp) + _TINY)
    lvi = jnp.sum(lsp[:, :, :16], axis=2) + w_ref[...][:, :1]
    lvb = jnp.sum(lsp[:, :, 16:], axis=2) + w_ref[...][:, :1]
    mi = jnp.max(lvi, axis=0, keepdims=True)
    mb = jnp.max(lvb, axis=0, keepdims=True)
    ti = jnp.log(jnp.sum(jnp.exp(lvi - mi), axis=0)) + mi[0]
    tb = jnp.log(jnp.sum(jnp.exp(lvb - mb), axis=0)) + mb[0]
    log_p = ti - tb
    p_ref[...] = log_p[None, None]
    e_ref[...] = jnp.exp(log_p)[None, None]


def _make_sc_gather(M, N, Bsz, CH, NCHUNK, PAIRS_PER_W):
    mesh = plsc.VectorSubcoreMesh(core_axis_name="c", subcore_axis_name="s")
    NSUB = CH // 128

    @functools.partial(
        pl.kernel,
        mesh=mesh,
        compiler_params=pltpu.CompilerParams(use_tc_tiling_on_sc=False),
        out_type=[
            jax.ShapeDtypeStruct((M * Bsz, 32), jnp.float32),
            jax.ShapeDtypeStruct((M * Bsz, 32), jnp.float32),
            jax.ShapeDtypeStruct((M * Bsz, 32), jnp.float32),
        ],
        scratch_types=[
            pltpu.VMEM((CH // 128, 128), jnp.int32),
            pltpu.VMEM((CH // 128, 128), jnp.int32),
            pltpu.VMEM((CH, 32), jnp.float32),
            pltpu.VMEM((CH, 32), jnp.float32),
            pltpu.VMEM((CH, 32), jnp.float32),
            pltpu.SemaphoreType.DMA,
        ],
    )
    def sc_gather(idx0_hbm, idx1_hbm, table_hbm, outa_hbm, outb_hbm,
                  outd_hbm, idxa_v, idxb_v, rowsa_v, rowsb_v, dpk_v, sem):
        cid = lax.axis_index("c")
        sid = lax.axis_index("s")
        wid = sid * 2 + cid
        m = wid // 4
        q = wid % 4
        mN = m * N

        for chunk in range(NCHUNK):
            p0 = q * PAIRS_PER_W + chunk * CH
            r0 = pl.multiple_of(p0 // 128, 8)
            pltpu.sync_copy(idx0_hbm.at[pl.ds(r0, CH // 128)], idxa_v)
            pltpu.sync_copy(idx1_hbm.at[pl.ds(r0, CH // 128)], idxb_v)

            def _adjust(i, _):
                k = i // 8
                j = (i % 8) * 16
                idxa_v[k, pl.ds(j, 16)] = idxa_v[k, pl.ds(j, 16)] + mN
                idxb_v[k, pl.ds(j, 16)] = idxb_v[k, pl.ds(j, 16)] + mN
                return 0

            lax.fori_loop(0, (CH // 128) * 8, _adjust, 0)

            copies = []
            for k in range(NSUB):
                copies.append(pltpu.async_copy(
                    table_hbm.at[idxa_v.at[k]],
                    rowsa_v.at[pl.ds(k * 128, 128)], sem))
                copies.append(pltpu.async_copy(
                    table_hbm.at[idxb_v.at[k]],
                    rowsb_v.at[pl.ds(k * 128, 128)], sem))
            for c in copies:
                c.wait()

            def _compute(i, _):
                za = rowsa_v[i, pl.ds(0, 16)]
                Za = rowsa_v[i, pl.ds(16, 16)]
                zb = rowsb_v[i, pl.ds(0, 16)]
                Zb = rowsb_v[i, pl.ds(16, 16)]
                za = jnp.minimum(jnp.maximum(za, 0.0), 1.0)
                Za = jnp.minimum(jnp.maximum(Za, 0.0), 1.0)
                zb = jnp.minimum(jnp.maximum(zb, 0.0), 1.0)
                Zb = jnp.minimum(jnp.maximum(Zb, 0.0), 1.0)
                rowsa_v[i, pl.ds(0, 16)] = za
                rowsa_v[i, pl.ds(16, 16)] = Za
                rowsb_v[i, pl.ds(0, 16)] = zb
                rowsb_v[i, pl.ds(16, 16)] = Zb
                dpk_v[i, pl.ds(0, 16)] = (
                    jnp.minimum(Za, Zb) - jnp.maximum(za, zb))
                dpk_v[i, pl.ds(16, 16)] = Zb - zb
                return 0

            lax.fori_loop(0, CH, _compute, 0)

            o0 = pl.multiple_of(m * Bsz + p0, 8)
            pltpu.sync_copy(rowsa_v, outa_hbm.at[pl.ds(o0, CH)])
            pltpu.sync_copy(rowsb_v, outb_hbm.at[pl.ds(o0, CH)])
            pltpu.sync_copy(dpk_v, outd_hbm.at[pl.ds(o0, CH)])

    return sc_gather


def kernel(box_indices, box_param, weights):
    M, N, _, D = box_param.shape
    Bsz = box_indices.shape[0]
    NB = 2000
    grid1 = N // NB
    PB = 2048
    grid2 = Bsz // PB

    log_universe_vol = float(D) * float(np.log(np.log1p(np.e) + _TINY))
    wadj = (weights - log_universe_vol).astype(jnp.float32)
    wcol = jnp.broadcast_to(wadj[:, None], (M, 128))
    wmax = jnp.max(weights)
    lsew = (jnp.log(jnp.sum(jnp.exp(weights - wmax))) + wmax).reshape(1)

    table32 = box_param.reshape(M, N, 2 * D)
    table_rows = box_param.reshape(M * N, 2 * D)
    idx0 = box_indices[:, 0].reshape(Bsz // 128, 128).astype(jnp.int32)
    idx1 = box_indices[:, 1].reshape(Bsz // 128, 128).astype(jnp.int32)

    PAIRS_PER_W = Bsz // 4
    CH = 1024
    NCHUNK = PAIRS_PER_W // CH
    sc_gather = _make_sc_gather(M, N, Bsz, CH, NCHUNK, PAIRS_PER_W)
    a32, b32, dpack = sc_gather(idx0, idx1, table_rows)

    unary2d = pl.pallas_call(
        _unary_body,
        grid=(grid1,),
        in_specs=[
            pl.BlockSpec((M, NB, 2 * D), lambda i: (0, i, 0)),
            pl.BlockSpec((M, 128), lambda i: (0, 0)),
            pl.BlockSpec(memory_space=pltpu.SMEM),
        ],
        out_specs=pl.BlockSpec((1, 1, NB), lambda i: (i, 0, 0)),
        out_shape=jax.ShapeDtypeStruct((grid1, 1, NB), jnp.float32),
    )(table32, wcol, lsew)
    log_unary_probs = unary2d.reshape(N)

    logp3, expp3 = pl.pallas_call(
        _pair_body,
        grid=(grid2,),
        in_specs=[
            pl.BlockSpec((M, PB, 32), lambda i: (0, i, 0)),
            pl.BlockSpec((M, 128), lambda i: (0, 0)),
        ],
        out_specs=[
            pl.BlockSpec((1, 1, PB), lambda i: (i, 0, 0)),
            pl.BlockSpec((1, 1, PB), lambda i: (i, 0, 0)),
        ],
        out_shape=[
            jax.ShapeDtypeStruct((grid2, 1, PB), jnp.float32),
            jax.ShapeDtypeStruct((grid2, 1, PB), jnp.float32),
        ],
    )(dpack.reshape(M, Bsz, 32), wcol)
    log_p = logp3.reshape(Bsz)
    exp_p = expp3.reshape(Bsz)

    A = a32.reshape(M, Bsz, 2, D)
    B = b32.reshape(M, Bsz, 2, D)
    return (log_unary_probs, box_param, A, B, log_p, exp_p)
